# Initial kernel scaffold; baseline (speedup 1.0000x reference)
import functools

import jax
import jax.numpy as jnp
from jax import lax
from jax.experimental import pallas as pl
from jax.experimental.pallas import tpu as pltpu
from jax.experimental.pallas import tpu_sc as plsc

V = 50000
E = 1600000
NMESH = 16
NC = 2
NS = 16
L = 16
NW = NC * NS
EPT = E // NW
S = 2000
NSC = EPT // S
G = S // L


def _rsqrt(a):
    i = plsc.bitcast(a, jnp.int32)
    i = jnp.int32(0x5F3759DF) - lax.shift_right_logical(i, 1)
    y = plsc.bitcast(i, jnp.float32)
    half = a * 0.5
    for _ in range(3):
        y = y * (1.5 - half * y * y)
    return y


def _tile_kernel(tabs_hbm, edges_hbm, e2m_hbm, out_s_hbm, out_c_hbm,
                 d2_sh, dt2_sh,
                 xtab, ttab, echunk, e2mc, d2c, dt2c, sbins, cbins):
    cid = lax.axis_index("c")
    sid = lax.axis_index("s")
    wid = sid * NC + cid
    ebase = wid * EPT

    zeros = jnp.zeros((L,), jnp.float32)
    for i in range(NMESH):
        sbins[pl.ds(i * L, L)] = zeros
        cbins[pl.ds(i * L, L)] = zeros

    iota = lax.iota(jnp.int32, L)
    pos0 = iota * 2

    for p in range(3):
        pltpu.sync_copy(tabs_hbm.at[p], xtab)
        pltpu.sync_copy(tabs_hbm.at[p + 3], ttab)

        def superchunk(s, carry, p=p):
            pltpu.sync_copy(
                edges_hbm.at[pl.ds((ebase + s * S) * 2, 2 * S)], echunk)
            if p > 0:
                pltpu.sync_copy(d2_sh.at[sid, pl.ds(s * S, S)], d2c)
                pltpu.sync_copy(dt2_sh.at[sid, pl.ds(s * S, S)], dt2c)
            if p == 2:
                pltpu.sync_copy(e2m_hbm.at[pl.ds(ebase + s * S, S)], e2mc)

            def group(g, carry2, p=p):
                base = g * (2 * L)
                src = plsc.load_gather(echunk, [base + pos0])
                dst = plsc.load_gather(echunk, [base + pos0 + 1])
                dx = (plsc.load_gather(xtab, [src])
                      - plsc.load_gather(xtab, [dst]))
                dxt = (plsc.load_gather(ttab, [src])
                       - plsc.load_gather(ttab, [dst]))
                sl = pl.ds(g * L, L)
                if p == 0:
                    d2c[sl] = dx * dx
                    dt2c[sl] = dxt * dxt
                elif p == 1:
                    d2c[sl] = d2c[sl] + dx * dx
                    dt2c[sl] = dt2c[sl] + dxt * dxt
                else:
                    d2 = d2c[sl] + dx * dx
                    dt2 = dt2c[sl] + dxt * dxt
                    d = jnp.where(d2 > 0.0, d2 * _rsqrt(d2), 0.0)
                    dt = jnp.where(dt2 > 0.0, dt2 * _rsqrt(dt2), 0.0)
                    diff = d - dt
                    x = diff * diff
                    bidx = plsc.load_gather(e2mc, [g * L + iota]) * L + iota
                    plsc.addupdate_scatter(sbins, [bidx], x)
                    plsc.addupdate_scatter(cbins, [bidx],
                                           jnp.full((L,), 1.0, jnp.float32))
                return carry2

            lax.fori_loop(0, G, group, 0)

            if p < 2:
                pltpu.sync_copy(d2c, d2_sh.at[sid, pl.ds(s * S, S)])
                pltpu.sync_copy(dt2c, dt2_sh.at[sid, pl.ds(s * S, S)])
            return carry

        lax.fori_loop(0, NSC, superchunk, 0)

    pltpu.sync_copy(sbins, out_s_hbm.at[wid])
    pltpu.sync_copy(cbins, out_c_hbm.at[wid])


@jax.jit
def _run(tabs, edges_flat, e2m):
    mesh = plsc.VectorSubcoreMesh(core_axis_name="c", subcore_axis_name="s")
    f = pl.kernel(
        _tile_kernel,
        mesh=mesh,
        out_type=[
            jax.ShapeDtypeStruct((NW, NMESH * L), jnp.float32),
            jax.ShapeDtypeStruct((NW, NMESH * L), jnp.float32),
        ],
        scratch_types=[
            pltpu.VMEM_SHARED((NS, EPT), jnp.float32),
            pltpu.VMEM_SHARED((NS, EPT), jnp.float32),
            pltpu.VMEM((V,), jnp.float32),
            pltpu.VMEM((V,), jnp.float32),
            pltpu.VMEM((2 * S,), jnp.int32),
            pltpu.VMEM((S,), jnp.int32),
            pltpu.VMEM((S,), jnp.float32),
            pltpu.VMEM((S,), jnp.float32),
            pltpu.VMEM((NMESH * L,), jnp.float32),
            pltpu.VMEM((NMESH * L,), jnp.float32),
        ],
    )
    return f(tabs, edges_flat, e2m)


def kernel(verts_packed, edges_packed, edge_to_mesh_idx, verts_packed_t,
           edges_packed_t, num_meshes):
    tabs = jnp.concatenate(
        [jnp.transpose(verts_packed), jnp.transpose(verts_packed_t)], axis=0)
    edges_flat = edges_packed.reshape(-1)
    out_s, out_c = _run(tabs, edges_flat, edge_to_mesh_idx)
    s_m = out_s.reshape(NW, NMESH, L).sum(axis=(0, 2))
    c_m = out_c.reshape(NW, NMESH, L).sum(axis=(0, 2))
    loss = jnp.where(c_m > 0, s_m / c_m, 0.0).sum() / num_meshes
    return loss


# speedup vs baseline: 18.6254x; 18.6254x over previous
import functools

import jax
import jax.numpy as jnp
from jax import lax
from jax.experimental import pallas as pl
from jax.experimental.pallas import tpu as pltpu
from jax.experimental.pallas import tpu_sc as plsc

V = 50000
E = 1600000
NMESH = 16
NC = 2
NS = 16
L = 16
NW = NC * NS
EPT = E // NW
S = 2000
NSC = EPT // S
G = S // L


def _rsqrt(a):
    i = plsc.bitcast(a, jnp.int32)
    i = jnp.int32(0x5F3759DF) - lax.shift_right_logical(i, 1)
    y = plsc.bitcast(i, jnp.float32)
    half = a * 0.5
    for _ in range(3):
        y = y * (1.5 - half * y * y)
    return y


def _tile_kernel(tabs_hbm, edges_hbm, e2m_hbm, out_s_hbm, out_c_hbm,
                 d2_sh, dt2_sh,
                 xtab, ttab, echunk, e2mc, d2c, dt2c, sbins, cbins):
    cid = lax.axis_index("c")
    sid = lax.axis_index("s")
    wid = sid * NC + cid
    ebase = wid * EPT

    zeros = jnp.zeros((L,), jnp.float32)
    for i in range(NMESH):
        sbins[pl.ds(i * L, L)] = zeros
        cbins[pl.ds(i * L, L)] = zeros

    iota = lax.iota(jnp.int32, L)
    pos0 = iota * 2

    for p in range(3):
        pltpu.sync_copy(tabs_hbm.at[p], xtab)
        pltpu.sync_copy(tabs_hbm.at[p + 3], ttab)

        def superchunk(s, carry, p=p):
            pltpu.sync_copy(
                edges_hbm.at[pl.ds((ebase + s * S) * 2, 2 * S)], echunk)
            if p > 0:
                pltpu.sync_copy(d2_sh.at[pl.ds(ebase + s * S, S)], d2c)
                pltpu.sync_copy(dt2_sh.at[pl.ds(ebase + s * S, S)], dt2c)
            if p == 2:
                pltpu.sync_copy(e2m_hbm.at[pl.ds(ebase + s * S, S)], e2mc)

            def group(g, carry2, p=p):
                base = g * (2 * L)
                src = plsc.load_gather(echunk, [base + pos0])
                dst = plsc.load_gather(echunk, [base + pos0 + 1])
                dx = (plsc.load_gather(xtab, [src])
                      - plsc.load_gather(xtab, [dst]))
                dxt = (plsc.load_gather(ttab, [src])
                       - plsc.load_gather(ttab, [dst]))
                sl = pl.ds(g * L, L)
                if p == 0:
                    d2c[sl] = dx * dx
                    dt2c[sl] = dxt * dxt
                elif p == 1:
                    d2c[sl] = d2c[sl] + dx * dx
                    dt2c[sl] = dt2c[sl] + dxt * dxt
                else:
                    d2 = d2c[sl] + dx * dx
                    dt2 = dt2c[sl] + dxt * dxt
                    d = jnp.where(d2 > 0.0, d2 * _rsqrt(d2), 0.0)
                    dt = jnp.where(dt2 > 0.0, dt2 * _rsqrt(dt2), 0.0)
                    diff = d - dt
                    x = diff * diff
                    bidx = plsc.load_gather(e2mc, [g * L + iota]) * L + iota
                    plsc.addupdate_scatter(sbins, [bidx], x)
                    plsc.addupdate_scatter(cbins, [bidx],
                                           jnp.full((L,), 1.0, jnp.float32))
                return carry2

            lax.fori_loop(0, G, group, 0)

            if p < 2:
                pltpu.sync_copy(d2c, d2_sh.at[pl.ds(ebase + s * S, S)])
                pltpu.sync_copy(dt2c, dt2_sh.at[pl.ds(ebase + s * S, S)])
            return carry

        lax.fori_loop(0, NSC, superchunk, 0)

    pltpu.sync_copy(sbins, out_s_hbm.at[wid])
    pltpu.sync_copy(cbins, out_c_hbm.at[wid])


@jax.jit
def _run(tabs, edges_flat, e2m):
    mesh = plsc.VectorSubcoreMesh(core_axis_name="c", subcore_axis_name="s")
    f = pl.kernel(
        _tile_kernel,
        mesh=mesh,
        compiler_params=pltpu.CompilerParams(needs_layout_passes=False),
        out_type=[
            jax.ShapeDtypeStruct((NW, NMESH * L), jnp.float32),
            jax.ShapeDtypeStruct((NW, NMESH * L), jnp.float32),
        ],
        scratch_types=[
            pltpu.HBM((E,), jnp.float32),
            pltpu.HBM((E,), jnp.float32),
            pltpu.VMEM((V,), jnp.float32),
            pltpu.VMEM((V,), jnp.float32),
            pltpu.VMEM((2 * S,), jnp.int32),
            pltpu.VMEM((S,), jnp.int32),
            pltpu.VMEM((S,), jnp.float32),
            pltpu.VMEM((S,), jnp.float32),
            pltpu.VMEM((NMESH * L,), jnp.float32),
            pltpu.VMEM((NMESH * L,), jnp.float32),
        ],
    )
    return f(tabs, edges_flat, e2m)


def kernel(verts_packed, edges_packed, edge_to_mesh_idx, verts_packed_t,
           edges_packed_t, num_meshes):
    tabs = jnp.concatenate(
        [jnp.transpose(verts_packed), jnp.transpose(verts_packed_t)], axis=0)
    edges_flat = edges_packed.reshape(-1)
    out_s, out_c = _run(tabs, edges_flat, edge_to_mesh_idx)
    s_m = out_s.reshape(NW, NMESH, L).sum(axis=(0, 2))
    c_m = out_c.reshape(NW, NMESH, L).sum(axis=(0, 2))
    loss = jnp.where(c_m > 0, s_m / c_m, 0.0).sum() / num_meshes
    return loss
